# baseline (device time: 13877 ns/iter reference)
import jax
import jax.numpy as jnp
from jax import lax
from jax.experimental import pallas as pl
from jax.experimental.pallas import tpu as pltpu

N_DEV = 16
C_GLOBAL = 2048
EPS = 1e-5


def kernel(x, t_emb, W_scale, W_shift):
    b, s, c_loc = x.shape

    def body(x_ref, t_ref, ws_ref, wh_ref, out_ref,
             acc_ref, recv_ref, send_sems, recv_sems):
        my = lax.axis_index("i")

        barrier = pltpu.get_barrier_semaphore()
        for k in range(1, N_DEV):
            peer = lax.rem(my + k, N_DEV)
            pl.semaphore_signal(barrier, inc=1, device_id=(peer,),
                                device_id_type=pl.DeviceIdType.MESH)
        pl.semaphore_wait(barrier, N_DEV - 1)

        xv = x_ref[...]
        sums = jnp.sum(xv, axis=-1)
        sqs = jnp.sum(xv * xv, axis=-1)
        acc_ref[...] = jnp.concatenate([sums, sqs], axis=0)

        rdmas = []
        for k in range(1, N_DEV):
            peer = lax.rem(my + k, N_DEV)
            rdma = pltpu.make_async_remote_copy(
                src_ref=acc_ref,
                dst_ref=recv_ref.at[k],
                send_sem=send_sems.at[k],
                recv_sem=recv_sems.at[k],
                device_id=(peer,),
                device_id_type=pl.DeviceIdType.MESH,
            )
            rdma.start()
            rdmas.append(rdma)

        scale = jnp.dot(t_ref[...], ws_ref[...],
                        preferred_element_type=jnp.float32)
        shift = jnp.dot(t_ref[...], wh_ref[...],
                        preferred_element_type=jnp.float32)

        total = acc_ref[...]
        for k in range(1, N_DEV):
            rdmas[k - 1].wait_recv()
            total = total + recv_ref[k]
        for r in rdmas:
            r.wait_send()

        mean = total[:b, :] * (1.0 / C_GLOBAL)
        ex2 = total[b:, :] * (1.0 / C_GLOBAL)
        var = ex2 - mean * mean
        inv = lax.rsqrt(var + EPS)
        h = (xv - mean[:, :, None]) * inv[:, :, None]
        out_ref[...] = h * (1.0 + scale[:, None, :]) + shift[:, None, :]

    return pl.pallas_call(
        body,
        out_shape=jax.ShapeDtypeStruct((b, s, c_loc), jnp.float32),
        in_specs=[pl.BlockSpec(memory_space=pltpu.VMEM)] * 4,
        out_specs=pl.BlockSpec(memory_space=pltpu.VMEM),
        scratch_shapes=[
            pltpu.VMEM((2 * b, s), jnp.float32),
            pltpu.VMEM((N_DEV, 2 * b, s), jnp.float32),
            pltpu.SemaphoreType.DMA((N_DEV,)),
            pltpu.SemaphoreType.DMA((N_DEV,)),
        ],
        compiler_params=pltpu.CompilerParams(collective_id=0),
    )(x, t_emb, W_scale, W_shift)


# device time: 12963 ns/iter; 1.0705x vs baseline; 1.0705x over previous
import jax
import jax.numpy as jnp
from jax import lax
from jax.experimental import pallas as pl
from jax.experimental.pallas import tpu as pltpu

N_DEV = 16
C_GLOBAL = 2048
EPS = 1e-5


def kernel(x, t_emb, W_scale, W_shift):
    b, s, c_loc = x.shape

    def body(x_ref, t_ref, ws_ref, wh_ref, out_ref,
             acc_ref, recv_ref, send_sems, recv_sems):
        my = lax.axis_index("i")

        barrier = pltpu.get_barrier_semaphore()
        for k in range(1, N_DEV):
            peer = lax.rem(my + k, N_DEV)
            pl.semaphore_signal(barrier, inc=1, device_id=(peer,),
                                device_id_type=pl.DeviceIdType.MESH)

        xv = x_ref[...]
        sums = jnp.sum(xv, axis=-1)
        sqs = jnp.sum(xv * xv, axis=-1)
        acc_ref[...] = jnp.concatenate([sums, sqs], axis=0)

        pl.semaphore_wait(barrier, N_DEV - 1)

        rdmas = []
        for k in range(1, N_DEV):
            peer = lax.rem(my + k, N_DEV)
            rdma = pltpu.make_async_remote_copy(
                src_ref=acc_ref,
                dst_ref=recv_ref.at[k],
                send_sem=send_sems.at[k],
                recv_sem=recv_sems.at[k],
                device_id=(peer,),
                device_id_type=pl.DeviceIdType.MESH,
            )
            rdma.start()
            rdmas.append(rdma)

        scale = jnp.dot(t_ref[...], ws_ref[...],
                        preferred_element_type=jnp.float32)
        shift = jnp.dot(t_ref[...], wh_ref[...],
                        preferred_element_type=jnp.float32)

        for r in rdmas:
            r.wait_recv()
        total = acc_ref[...]
        for k in range(1, N_DEV):
            total = total + recv_ref[k]
        for r in rdmas:
            r.wait_send()

        total_t = total.T * (1.0 / C_GLOBAL)
        one_scale = 1.0 + scale
        for bi in range(b):
            mean_c = total_t[:, bi:bi + 1]
            ex2_c = total_t[:, b + bi:b + bi + 1]
            inv_c = lax.rsqrt(ex2_c - mean_c * mean_c + EPS)
            h = (xv[bi] - mean_c) * inv_c
            out_ref[bi] = (h * one_scale[bi:bi + 1, :]
                           + shift[bi:bi + 1, :])

    return pl.pallas_call(
        body,
        out_shape=jax.ShapeDtypeStruct((b, s, c_loc), jnp.float32),
        in_specs=[pl.BlockSpec(memory_space=pltpu.VMEM)] * 4,
        out_specs=pl.BlockSpec(memory_space=pltpu.VMEM),
        scratch_shapes=[
            pltpu.VMEM((2 * b, s), jnp.float32),
            pltpu.VMEM((N_DEV, 2 * b, s), jnp.float32),
            pltpu.SemaphoreType.DMA((N_DEV,)),
            pltpu.SemaphoreType.DMA((N_DEV,)),
        ],
        compiler_params=pltpu.CompilerParams(collective_id=0),
    )(x, t_emb, W_scale, W_shift)
